# baseline (speedup 1.0000x reference)
import functools

import jax
import jax.numpy as jnp
from jax import lax
from jax.experimental import pallas as pl
from jax.experimental.pallas import tpu as pltpu
from jax.experimental.pallas import tpu_sc as plsc

N_NODES = 10000
N_EDGES = 160000
D = 256
NQ = 4
FW = D // NQ
NP = NQ + 1

NPAD = 10240
ROWS_PER_SUB = NPAD // 16
EPAD = 163840
CW = 128
NCHUNKS = EPAD // CW
K = NCHUNKS // 32
NBUF = 4

_mesh = plsc.VectorSubcoreMesh(core_axis_name="c", subcore_axis_name="s")


@functools.partial(
    pl.kernel,
    mesh=_mesh,
    compiler_params=pltpu.CompilerParams(use_tc_tiling_on_sc=False),
    out_type=jax.ShapeDtypeStruct((2, NP, NPAD, FW), jnp.float32),
    scratch_types=[
        pltpu.VMEM((NQ * K, CW), jnp.int32),
        pltpu.VMEM((K, CW), jnp.int32),
        [pltpu.VMEM((CW, FW), jnp.float32) for _ in range(NBUF)],
        pltpu.VMEM((CW, FW), jnp.float32),
        pltpu.VMEM((CW, FW), jnp.float32),
        pltpu.VMEM_SHARED((NPAD, FW), jnp.float32),
        [pltpu.SemaphoreType.DMA for _ in range(NBUF)],
        [pltpu.SemaphoreType.DMA for _ in range(NBUF)],
        pltpu.SemaphoreType.DMA,
    ],
)
def _sc_aggregate(tbl, srcq_a, dst_a, psum,
                  idx_s, idx_d, rows, zrows, ones_v, acc_sh, sg, ss, swb):
    c = lax.axis_index("c")
    s = lax.axis_index("s")
    base = s * ROWS_PER_SUB
    wid = c * 16 + s

    def _fill_const(i, _):
        for l in range(FW // 16):
            zrows[i, pl.ds(l * 16, 16)] = jnp.zeros((16,), jnp.float32)
            ones_v[i, pl.ds(l * 16, 16)] = jnp.ones((16,), jnp.float32)
        return 0

    lax.fori_loop(0, CW, _fill_const, 0, unroll=False)

    def _zero_stripe():
        def _z(t, _):
            pltpu.sync_copy(zrows, acc_sh.at[pl.ds(base + t * CW, CW)])
            return 0
        lax.fori_loop(0, ROWS_PER_SUB // CW, _z, 0, unroll=False)

    def _writeback(p):
        for t in range(ROWS_PER_SUB // CW):
            b = t % 2
            if t >= 2:
                pltpu.make_async_copy(
                    rows[b], psum.at[c, p, pl.ds(base, CW)], swb).wait()
            pltpu.sync_copy(acc_sh.at[pl.ds(base + t * CW, CW)], rows[b])
            pltpu.async_copy(
                rows[b], psum.at[c, p, pl.ds(base + t * CW, CW)], swb)
        for t in range(2):
            pltpu.make_async_copy(
                rows[t], psum.at[c, p, pl.ds(base, CW)], swb).wait()

    pltpu.sync_copy(srcq_a.at[wid], idx_s)
    pltpu.sync_copy(dst_a.at[wid], idx_d)

    _zero_stripe()
    plsc.subcore_barrier()

    def _quarter(q, _):
        qbase = q * K
        for b in range(NBUF):
            pltpu.async_copy(tbl.at[idx_s.at[qbase + b]], rows[b], sg[b])

        def _ring_block(t, __):
            for b in range(NBUF):
                j = t * NBUF + b
                pltpu.make_async_copy(
                    tbl.at[idx_s.at[qbase]], rows[b], sg[b]).wait()
                pltpu.async_copy(
                    rows[b], acc_sh.at[idx_d.at[j]], ss[b], add=True)
                pltpu.make_async_copy(
                    rows[b], acc_sh.at[idx_d.at[0]], ss[b]).wait()

                @pl.when(t < K // NBUF - 1)
                def _():
                    pltpu.async_copy(
                        tbl.at[idx_s.at[qbase + j + NBUF]], rows[b], sg[b])
            return 0

        lax.fori_loop(0, K // NBUF, _ring_block, 0, unroll=False)
        plsc.subcore_barrier()
        _writeback(q)
        _zero_stripe()
        plsc.subcore_barrier()
        return 0

    lax.fori_loop(0, NQ, _quarter, 0, unroll=False)

    def _deg_block(t, _):
        for b in range(NBUF):
            j = t * NBUF + b
            pltpu.async_copy(ones_v, acc_sh.at[idx_d.at[j]], ss[b], add=True)
        for b in range(NBUF):
            pltpu.make_async_copy(
                ones_v, acc_sh.at[idx_d.at[0]], ss[b]).wait()
        return 0

    lax.fori_loop(0, K // NBUF, _deg_block, 0, unroll=False)
    plsc.subcore_barrier()
    _writeback(NQ)


BLK = 2000


def _tc_body(x_ref, ps, wst, wnt, b_ref, o_ref):
    deg = jnp.maximum(ps[0, NQ, :, 0:1] + ps[1, NQ, :, 0:1], 1.0)
    hn = jnp.concatenate(
        [ps[0, q] + ps[1, q] for q in range(NQ)], axis=1) / deg
    o_ref[...] = (
        jnp.dot(x_ref[...], wst[...], preferred_element_type=jnp.float32)
        + jnp.dot(hn, wnt[...], preferred_element_type=jnp.float32)
        + b_ref[...]
    )


def _tc_combine(x, psum, wst, wnt, b2d):
    return pl.pallas_call(
        _tc_body,
        grid=(N_NODES // BLK,),
        in_specs=[
            pl.BlockSpec((BLK, D), lambda i: (i, 0)),
            pl.BlockSpec((2, NP, BLK, FW), lambda i: (0, 0, i, 0)),
            pl.BlockSpec((D, D), lambda i: (0, 0)),
            pl.BlockSpec((D, D), lambda i: (0, 0)),
            pl.BlockSpec((1, D), lambda i: (0, 0)),
        ],
        out_specs=pl.BlockSpec((BLK, D), lambda i: (i, 0)),
        out_shape=jax.ShapeDtypeStruct((N_NODES, D), jnp.float32),
    )(x, psum, wst, wnt, b2d)


def kernel(x, edge_index, W_self, W_neigh, b):
    x = x.astype(jnp.float32)
    src = edge_index[0].astype(jnp.int32)
    dst = edge_index[1].astype(jnp.int32)

    tbl = x.reshape(N_NODES * NQ, FW)

    npad_e = EPAD - N_EDGES
    pad_dst = N_NODES + (
        jnp.arange(npad_e, dtype=jnp.int32) % (NPAD - N_NODES))
    src_p = jnp.concatenate(
        [src, jnp.zeros((npad_e,), jnp.int32)]).reshape(32, K, CW)
    dst_a = jnp.concatenate([dst, pad_dst]).reshape(32, K, CW)
    srcq_a = (
        src_p[:, None, :, :] * NQ
        + jnp.arange(NQ, dtype=jnp.int32)[None, :, None, None]
    ).reshape(32, NQ * K, CW)

    psum = _sc_aggregate(tbl, srcq_a, dst_a)

    return _tc_combine(x, psum, W_self.T, W_neigh.T, b.reshape(1, D))


# speedup vs baseline: 2.6469x; 2.6469x over previous
import functools

import jax
import jax.numpy as jnp
from jax import lax
from jax.experimental import pallas as pl
from jax.experimental.pallas import tpu as pltpu
from jax.experimental.pallas import tpu_sc as plsc

N_NODES = 10000
N_EDGES = 160000
D = 256
NQ = 4
FW = D // NQ
NP = NQ + 1

NPAD = 10240
ROWS_PER_SUB = NPAD // 16
EPAD = 163840
CW = 128
NCHUNKS = EPAD // CW
K = NCHUNKS // 32
NBUF = 4

_mesh = plsc.VectorSubcoreMesh(core_axis_name="c", subcore_axis_name="s")


@functools.partial(
    pl.kernel,
    mesh=_mesh,
    compiler_params=pltpu.CompilerParams(use_tc_tiling_on_sc=False),
    out_type=jax.ShapeDtypeStruct((2, NP, NPAD, FW), jnp.float32),
    scratch_types=[
        pltpu.VMEM((NQ * K, CW), jnp.int32),
        pltpu.VMEM((K, CW), jnp.int32),
        [pltpu.VMEM((CW, FW), jnp.float32) for _ in range(NBUF)],
        pltpu.VMEM((CW, FW), jnp.float32),
        pltpu.VMEM((CW, FW), jnp.float32),
        pltpu.VMEM_SHARED((NPAD, FW), jnp.float32),
        [pltpu.SemaphoreType.DMA for _ in range(NBUF)],
        [pltpu.SemaphoreType.DMA for _ in range(NBUF)],
        pltpu.SemaphoreType.DMA,
    ],
)
def _sc_aggregate(tbl, srcq_a, dst_a, psum,
                  idx_s, idx_d, rows, zrows, ones_v, acc_sh, sg, ss, swb):
    c = lax.axis_index("c")
    s = lax.axis_index("s")
    base = s * ROWS_PER_SUB
    wid = c * 16 + s

    def _fill_const(i, _):
        for l in range(FW // 16):
            zrows[i, pl.ds(l * 16, 16)] = jnp.zeros((16,), jnp.float32)
            ones_v[i, pl.ds(l * 16, 16)] = jnp.ones((16,), jnp.float32)
        return 0

    lax.fori_loop(0, CW, _fill_const, 0, unroll=False)

    def _zero_stripe():
        def _z(t, _):
            pltpu.sync_copy(zrows, acc_sh.at[pl.ds(base + t * CW, CW)])
            return 0
        lax.fori_loop(0, ROWS_PER_SUB // CW, _z, 0, unroll=False)

    def _writeback(p):
        for t in range(ROWS_PER_SUB // CW):
            b = t % 2
            if t >= 2:
                pltpu.make_async_copy(
                    rows[b], psum.at[c, p, pl.ds(base, CW)], swb).wait()
            pltpu.sync_copy(acc_sh.at[pl.ds(base + t * CW, CW)], rows[b])
            pltpu.async_copy(
                rows[b], psum.at[c, p, pl.ds(base + t * CW, CW)], swb)
        for t in range(2):
            pltpu.make_async_copy(
                rows[t], psum.at[c, p, pl.ds(base, CW)], swb).wait()

    with jax.named_scope("idx_load"):
        pltpu.sync_copy(srcq_a.at[wid], idx_s)
        pltpu.sync_copy(dst_a.at[wid], idx_d)

    with jax.named_scope("zero0"):
        _zero_stripe()
    plsc.subcore_barrier()

    def _quarter(q, _):
        qbase = q * K
        for b in range(NBUF):
            pltpu.async_copy(tbl.at[idx_s.at[qbase + b]], rows[b], sg[b])

        def _ring_block(t, __):
            for b in range(NBUF):
                j = t * NBUF + b
                pltpu.make_async_copy(
                    tbl.at[idx_s.at[qbase]], rows[b], sg[b]).wait()
                pltpu.async_copy(
                    rows[b], acc_sh.at[idx_d.at[j]], ss[b], add=True)
                pltpu.make_async_copy(
                    rows[b], acc_sh.at[idx_d.at[0]], ss[b]).wait()

                @pl.when(t < K // NBUF - 1)
                def _():
                    pltpu.async_copy(
                        tbl.at[idx_s.at[qbase + j + NBUF]], rows[b], sg[b])
            return 0

        with jax.named_scope("ring"):
            lax.fori_loop(0, K // NBUF, _ring_block, 0, unroll=False)
        with jax.named_scope("postbar"):
            plsc.subcore_barrier()
        with jax.named_scope("wb"):
            _writeback(q)
        with jax.named_scope("rezero"):
            _zero_stripe()
        plsc.subcore_barrier()
        return 0

    lax.fori_loop(0, NQ, _quarter, 0, unroll=False)

    def _deg_block(t, _):
        for b in range(NBUF):
            j = t * NBUF + b
            pltpu.async_copy(ones_v, acc_sh.at[idx_d.at[j]], ss[b], add=True)
        for b in range(NBUF):
            pltpu.make_async_copy(
                ones_v, acc_sh.at[idx_d.at[0]], ss[b]).wait()
        return 0

    with jax.named_scope("deg"):
        lax.fori_loop(0, K // NBUF, _deg_block, 0, unroll=False)
    plsc.subcore_barrier()
    with jax.named_scope("wbdeg"):
        _writeback(NQ)


BLK = 2000


def _tc_body(x_ref, ps, wst, wnt, b_ref, o_ref):
    deg = jnp.maximum(ps[0, NQ, :, 0:1] + ps[1, NQ, :, 0:1], 1.0)
    hn = jnp.concatenate(
        [ps[0, q] + ps[1, q] for q in range(NQ)], axis=1) / deg
    o_ref[...] = (
        jnp.dot(x_ref[...], wst[...], preferred_element_type=jnp.float32)
        + jnp.dot(hn, wnt[...], preferred_element_type=jnp.float32)
        + b_ref[...]
    )


def _tc_combine(x, psum, wst, wnt, b2d):
    return pl.pallas_call(
        _tc_body,
        grid=(N_NODES // BLK,),
        in_specs=[
            pl.BlockSpec((BLK, D), lambda i: (i, 0)),
            pl.BlockSpec((2, NP, BLK, FW), lambda i: (0, 0, i, 0)),
            pl.BlockSpec((D, D), lambda i: (0, 0)),
            pl.BlockSpec((D, D), lambda i: (0, 0)),
            pl.BlockSpec((1, D), lambda i: (0, 0)),
        ],
        out_specs=pl.BlockSpec((BLK, D), lambda i: (i, 0)),
        out_shape=jax.ShapeDtypeStruct((N_NODES, D), jnp.float32),
    )(x, psum, wst, wnt, b2d)


def kernel(x, edge_index, W_self, W_neigh, b):
    x = x.astype(jnp.float32)
    src = edge_index[0].astype(jnp.int32)
    dst = edge_index[1].astype(jnp.int32)

    tbl = x.reshape(N_NODES * NQ, FW)

    npad_e = EPAD - N_EDGES
    pad_dst = N_NODES + (
        jnp.arange(npad_e, dtype=jnp.int32) % (NPAD - N_NODES))
    pad_src = (jnp.arange(npad_e, dtype=jnp.int32) * 41) % N_NODES
    src_p = jnp.concatenate([src, pad_src]).reshape(32, K, CW)
    dst_a = jnp.concatenate([dst, pad_dst]).reshape(32, K, CW)
    srcq_a = (
        src_p[:, None, :, :] * NQ
        + jnp.arange(NQ, dtype=jnp.int32)[None, :, None, None]
    ).reshape(32, NQ * K, CW)

    psum = _sc_aggregate(tbl, srcq_a, dst_a)

    return _tc_combine(x, psum, W_self.T, W_neigh.T, b.reshape(1, D))
